# baseline (speedup 1.0000x reference)
import functools

import numpy as np
import jax
import jax.numpy as jnp
from jax.experimental import pallas as pl

_INT_MIN = np.int32(-2147483648)
_INV_SQRT2 = np.float32(1.0 / np.sqrt(2.0))


def _router_kernel(x_ref, w1_ref, b1_ref, w2_ref, b2_ref, scores_ref, mask_ref,
                   *, k):
    h = jnp.dot(x_ref[...], w1_ref[...], preferred_element_type=jnp.float32)
    h = h + b1_ref[...]
    h = 0.5 * h * (1.0 + jax.lax.erf(h * _INV_SQRT2))
    s = jnp.dot(h, w2_ref[...], preferred_element_type=jnp.float32)
    s = s + b2_ref[...]
    scores_ref[...] = s

    si = jax.lax.bitcast_convert_type(s, jnp.int32)
    w = jnp.where(si < 0, si ^ jnp.int32(0x7FFFFFFF), si)

    bm = s.shape[0]
    i16_min = jnp.int16(-32768)

    def row_count16(pred):
        t = jnp.where(pred, jnp.int16(1), jnp.int16(0))
        width = t.shape[1]
        while width > 512:
            half = width // 2
            t = t[:, :half] + t[:, half:]
            width = half
        return jnp.sum(t.astype(jnp.int32), axis=1, keepdims=True)

    w16 = jnp.right_shift(w, 16).astype(jnp.int16)

    def body_hi(i, c):
        bit_val = jnp.left_shift(jnp.int32(1), jnp.int32(15) - i)
        cand_off = c | bit_val
        cand = (cand_off ^ jnp.int32(32768)).astype(jnp.int16)
        cnt = row_count16(w16 >= cand)
        return jnp.where(cnt >= k, cand_off, c)

    c_hi_off = jax.lax.fori_loop(0, 16, body_hi,
                                 jnp.zeros((bm, 1), jnp.int32))
    p16 = (c_hi_off ^ jnp.int32(32768)).astype(jnp.int16)

    n_above = row_count16(w16 > p16)

    in_win = w16 == p16
    nchunk = w16.shape[1] // 128

    def fold(t, op):
        n = t.shape[1]
        while n > 1:
            half = n // 2
            t = op(t[:, :half, :], t[:, half:, :])
            n = half
        return t

    win3 = jnp.where(in_win, jnp.int16(1), jnp.int16(0)).reshape(bm, nchunk, 128)
    col_cnt = fold(win3, jnp.add).reshape(bm, 128)
    needs_full = jnp.max(col_cnt.astype(jnp.int32)) > 4

    def phase2_fast():
        lf = jnp.where(in_win,
                       (w & jnp.int32(0xFFFF)).astype(jnp.float32),
                       jnp.float32(-1.0))
        l3 = lf.reshape(bm, nchunk, 128)
        m1 = fold(l3, jnp.maximum)
        l3b = jnp.where(l3 == m1, jnp.float32(-1.0), l3)
        m2 = fold(l3b, jnp.maximum)
        l3c = jnp.where(l3b == m2, jnp.float32(-1.0), l3b)
        m3 = fold(l3c, jnp.maximum)
        l3d = jnp.where(l3c == m3, jnp.float32(-1.0), l3c)
        m4 = fold(l3d, jnp.maximum)
        compact = jnp.concatenate([m1, m2, m3, m4], axis=1).reshape(bm, 512)

        def body_lo(i, c):
            bit_val = jnp.left_shift(jnp.int32(1), jnp.int32(15) - i)
            cand_off = c | bit_val
            cnt = jnp.sum((compact >= cand_off.astype(jnp.float32))
                          .astype(jnp.int32), axis=1, keepdims=True)
            return jnp.where(n_above + cnt >= k, cand_off, c)

        return jax.lax.fori_loop(0, 16, body_lo,
                                 jnp.zeros((bm, 1), jnp.int32))

    def phase2_full():
        l16 = jnp.where(in_win, w.astype(jnp.int16) ^ i16_min, i16_min)

        def body_lo(i, c):
            bit_val = jnp.left_shift(jnp.int32(1), jnp.int32(15) - i)
            cand_off = c | bit_val
            cand = (cand_off ^ jnp.int32(32768)).astype(jnp.int16)
            cnt = row_count16(l16 >= cand)
            return jnp.where(n_above + cnt >= k, cand_off, c)

        return jax.lax.fori_loop(0, 16, body_lo,
                                 jnp.zeros((bm, 1), jnp.int32))

    c_lo_off = jax.lax.cond(needs_full, phase2_full, phase2_fast)

    thresh = (jnp.left_shift(c_hi_off - jnp.int32(32768), 16)
              | (c_lo_off & jnp.int32(0xFFFF)))
    mask_ref[...] = (w >= thresh).astype(jnp.float32)


def kernel(hidden_state, W1, b1, W2, b2):
    B, S, H = hidden_state.shape
    BOT = W1.shape[1]
    F = W2.shape[1]
    M = B * S
    k = max(1, int(F * (1.0 - 0.9)))

    BM = 128
    grid = (M // BM,)

    x = hidden_state.reshape(M, H)
    scores, mask = pl.pallas_call(
        functools.partial(_router_kernel, k=k),
        grid=grid,
        in_specs=[
            pl.BlockSpec((BM, H), lambda i: (i, 0)),
            pl.BlockSpec((H, BOT), lambda i: (0, 0)),
            pl.BlockSpec((1, BOT), lambda i: (0, 0)),
            pl.BlockSpec((BOT, F), lambda i: (0, 0)),
            pl.BlockSpec((1, F), lambda i: (0, 0)),
        ],
        out_specs=[
            pl.BlockSpec((BM, F), lambda i: (i, 0)),
            pl.BlockSpec((BM, F), lambda i: (i, 0)),
        ],
        out_shape=[
            jax.ShapeDtypeStruct((M, F), jnp.float32),
            jax.ShapeDtypeStruct((M, F), jnp.float32),
        ],
    )(x, W1, b1.reshape(1, BOT), W2, b2.reshape(1, F))
    return scores.reshape(B, S, F), mask.reshape(B, S, F)


# speedup vs baseline: 1.1613x; 1.1613x over previous
import functools

import numpy as np
import jax
import jax.numpy as jnp
from jax.experimental import pallas as pl
from jax.experimental.pallas import tpu as pltpu

_INV_SQRT2 = np.float32(1.0 / np.sqrt(2.0))


def _router_kernel(x_ref, w1_ref, b1_ref, w2_ref, b2_ref, scores_ref, mask_ref,
                   s_scratch, *, k, nb):
    step = pl.program_id(0)

    @pl.when(step > 0)
    def _select():
        w = s_scratch[...]
        bm = w.shape[0]
        i16_min = jnp.int16(-32768)

        def row_count16(pred):
            t = jnp.where(pred, jnp.int16(1), jnp.int16(0))
            width = t.shape[1]
            while width > 512:
                half = width // 2
                t = t[:, :half] + t[:, half:]
                width = half
            return jnp.sum(t.astype(jnp.int32), axis=1, keepdims=True)

        w16 = jnp.right_shift(w, 16).astype(jnp.int16)

        def body_hi(i, c):
            bit_val = jnp.left_shift(jnp.int32(1), jnp.int32(15) - i)
            cand_off = c | bit_val
            cand = (cand_off ^ jnp.int32(32768)).astype(jnp.int16)
            cnt = row_count16(w16 >= cand)
            return jnp.where(cnt >= k, cand_off, c)

        c_hi_off = jax.lax.fori_loop(0, 16, body_hi,
                                     jnp.zeros((bm, 1), jnp.int32))
        p16 = (c_hi_off ^ jnp.int32(32768)).astype(jnp.int16)

        n_above = row_count16(w16 > p16)

        l16 = jnp.where(w16 == p16, w.astype(jnp.int16) ^ i16_min, i16_min)

        def body_lo(i, c):
            bit_val = jnp.left_shift(jnp.int32(1), jnp.int32(15) - i)
            cand_off = c | bit_val
            cand = (cand_off ^ jnp.int32(32768)).astype(jnp.int16)
            cnt = row_count16(l16 >= cand)
            return jnp.where(n_above + cnt >= k, cand_off, c)

        c_lo_off = jax.lax.fori_loop(0, 16, body_lo,
                                     jnp.zeros((bm, 1), jnp.int32))

        thresh = (jnp.left_shift(c_hi_off - jnp.int32(32768), 16)
                  | (c_lo_off & jnp.int32(0xFFFF)))
        mask_ref[...] = (w >= thresh).astype(jnp.float32)

    @pl.when(step < nb)
    def _matmul():
        h = jnp.dot(x_ref[...], w1_ref[...],
                    preferred_element_type=jnp.float32)
        h = h + b1_ref[...]
        h = 0.5 * h * (1.0 + jax.lax.erf(h * _INV_SQRT2))
        s = jnp.dot(h, w2_ref[...], preferred_element_type=jnp.float32)
        s = s + b2_ref[...]
        scores_ref[...] = s
        si = jax.lax.bitcast_convert_type(s, jnp.int32)
        s_scratch[...] = jnp.where(si < 0, si ^ jnp.int32(0x7FFFFFFF), si)


def kernel(hidden_state, W1, b1, W2, b2):
    B, S, H = hidden_state.shape
    BOT = W1.shape[1]
    F = W2.shape[1]
    M = B * S
    k = max(1, int(F * (1.0 - 0.9)))

    BM = 128
    nb = M // BM
    grid = (nb + 1,)

    x = hidden_state.reshape(M, H)
    scores, mask = pl.pallas_call(
        functools.partial(_router_kernel, k=k, nb=nb),
        grid=grid,
        in_specs=[
            pl.BlockSpec((BM, H), lambda i: (jnp.minimum(i, nb - 1), 0)),
            pl.BlockSpec((H, BOT), lambda i: (0, 0)),
            pl.BlockSpec((1, BOT), lambda i: (0, 0)),
            pl.BlockSpec((BOT, F), lambda i: (0, 0)),
            pl.BlockSpec((1, F), lambda i: (0, 0)),
        ],
        out_specs=[
            pl.BlockSpec((BM, F), lambda i: (jnp.minimum(i, nb - 1), 0)),
            pl.BlockSpec((BM, F), lambda i: (jnp.maximum(i - 1, 0), 0)),
        ],
        out_shape=[
            jax.ShapeDtypeStruct((M, F), jnp.float32),
            jax.ShapeDtypeStruct((M, F), jnp.float32),
        ],
        scratch_shapes=[pltpu.VMEM((BM, F), jnp.int32)],
    )(x, W1, b1.reshape(1, BOT), W2, b2.reshape(1, F))
    return scores.reshape(B, S, F), mask.reshape(B, S, F)


# speedup vs baseline: 1.3268x; 1.1426x over previous
import functools

import numpy as np
import jax
import jax.numpy as jnp
from jax.experimental import pallas as pl

_INV_SQRT2 = np.float32(1.0 / np.sqrt(2.0))


def _router_kernel(x_ref, w1_ref, b1_ref, w2_ref, b2_ref, scores_ref, mask_ref,
                   *, k):
    h = jnp.dot(x_ref[...], w1_ref[...], preferred_element_type=jnp.float32)
    h = h + b1_ref[...]
    h = 0.5 * h * (1.0 + jax.lax.erf(h * _INV_SQRT2))
    s = jnp.dot(h, w2_ref[...], preferred_element_type=jnp.float32)
    s = s + b2_ref[...]
    scores_ref[...] = s

    si = jax.lax.bitcast_convert_type(s, jnp.int32)
    w = jnp.where(si < 0, si ^ jnp.int32(0x7FFFFFFF), si)

    bm = s.shape[0]
    i16_min = jnp.int16(-32768)

    def row_count16(pred):
        t = jnp.where(pred, jnp.int16(1), jnp.int16(0))
        width = t.shape[1]
        while width > 512:
            half = width // 2
            t = t[:, :half] + t[:, half:]
            width = half
        return jnp.sum(t.astype(jnp.int32), axis=1, keepdims=True)

    w16 = jnp.right_shift(w, 16).astype(jnp.int16)

    def body_hi(i, c):
        bit_val = jnp.left_shift(jnp.int32(1), jnp.int32(15) - i)
        cand_off = c | bit_val
        cand = (cand_off ^ jnp.int32(32768)).astype(jnp.int16)
        cnt = row_count16(w16 >= cand)
        return jnp.where(cnt >= k, cand_off, c)

    c_hi_off = jax.lax.fori_loop(0, 16, body_hi,
                                 jnp.zeros((bm, 1), jnp.int32))
    p16 = (c_hi_off ^ jnp.int32(32768)).astype(jnp.int16)

    n_above = row_count16(w16 > p16)

    l16 = jnp.where(w16 == p16, w.astype(jnp.int16) ^ i16_min, i16_min)

    def body_lo(i, c):
        bit_val = jnp.left_shift(jnp.int32(1), jnp.int32(15) - i)
        cand_off = c | bit_val
        cand = (cand_off ^ jnp.int32(32768)).astype(jnp.int16)
        cnt = row_count16(l16 >= cand)
        return jnp.where(n_above + cnt >= k, cand_off, c)

    c_lo_off = jax.lax.fori_loop(0, 16, body_lo,
                                 jnp.zeros((bm, 1), jnp.int32))

    thresh = (jnp.left_shift(c_hi_off - jnp.int32(32768), 16)
              | (c_lo_off & jnp.int32(0xFFFF)))
    mask_ref[...] = (w >= thresh).astype(jnp.float32)


def kernel(hidden_state, W1, b1, W2, b2):
    B, S, H = hidden_state.shape
    BOT = W1.shape[1]
    F = W2.shape[1]
    M = B * S
    k = max(1, int(F * (1.0 - 0.9)))

    BM = 256
    grid = (M // BM,)

    x = hidden_state.reshape(M, H)
    scores, mask = pl.pallas_call(
        functools.partial(_router_kernel, k=k),
        grid=grid,
        in_specs=[
            pl.BlockSpec((BM, H), lambda i: (i, 0)),
            pl.BlockSpec((H, BOT), lambda i: (0, 0)),
            pl.BlockSpec((1, BOT), lambda i: (0, 0)),
            pl.BlockSpec((BOT, F), lambda i: (0, 0)),
            pl.BlockSpec((1, F), lambda i: (0, 0)),
        ],
        out_specs=[
            pl.BlockSpec((BM, F), lambda i: (i, 0)),
            pl.BlockSpec((BM, F), lambda i: (i, 0)),
        ],
        out_shape=[
            jax.ShapeDtypeStruct((M, F), jnp.float32),
            jax.ShapeDtypeStruct((M, F), jnp.float32),
        ],
    )(x, W1, b1.reshape(1, BOT), W2, b2.reshape(1, F))
    return scores.reshape(B, S, F), mask.reshape(B, S, F)
